# Initial kernel scaffold; baseline (speedup 1.0000x reference)
import jax
import jax.numpy as jnp
from jax.experimental import pallas as pl


def kernel(z, W_in, b_in, W0, a0, g0, be0, W1, a1, g1, be1):
    raise NotImplementedError("write your pallas kernel here")


# speedup vs baseline: 5.3394x; 5.3394x over previous
import functools

import jax
import jax.numpy as jnp
from jax import lax
from jax.experimental import pallas as pl

B, N, D, H = 4, 256, 128, 128
ALPHA = 0.2
K_NN = 8
NEG_BIG = -9e15


def _gat_layer(x, adj_mask, W, a_full, g, be, apply_relu):
    Wh = lax.dot_general(x, W, (((1,), (1,)), ((), ())),
                         preferred_element_type=jnp.float32)
    a1 = a_full[:, :H]
    a2 = a_full[:, H:]
    f1 = jnp.sum(Wh * a1, axis=1)
    f2 = jnp.sum(Wh * a2, axis=1)
    e = f1[:, None] + f2[None, :]
    e = jnp.where(e >= 0, e, ALPHA * e)
    att = jnp.where(adj_mask, e, NEG_BIG)
    m = jnp.max(att, axis=1, keepdims=True)
    ex = jnp.exp(att - m)
    p = ex / jnp.sum(ex, axis=1, keepdims=True)
    h = lax.dot_general(p, Wh, (((1,), (0,)), ((), ())),
                        preferred_element_type=jnp.float32)
    mu = jnp.mean(h, axis=1, keepdims=True)
    var = jnp.mean((h - mu) ** 2, axis=1, keepdims=True)
    y = (h - mu) / jnp.sqrt(var + 1e-5) * g + be
    if apply_relu:
        y = jnp.maximum(y, 0.0)
    return x + y


def _body(z_ref, Win_ref, bin_ref, W0_ref, a0_ref, g0_ref, be0_ref,
          W1_ref, a1_ref, g1_ref, be1_ref, out_ref):
    z = z_ref[0]

    G = lax.dot_general(z, z, (((1,), (1,)), ((), ())),
                        preferred_element_type=jnp.float32,
                        precision=lax.Precision.HIGHEST)
    s = jnp.sum(z * z, axis=1)
    d = s[:, None] + s[None, :] - 2.0 * G
    iota_i = lax.broadcasted_iota(jnp.int32, (N, N), 0)
    iota_j = lax.broadcasted_iota(jnp.int32, (N, N), 1)
    d = jnp.where(iota_i == iota_j, 1e6, d)

    adj = jnp.zeros((N, N), dtype=jnp.bool_)
    dd = d
    for _ in range(K_NN):
        m = jnp.min(dd, axis=1, keepdims=True)
        cand = jnp.where(dd == m, iota_j, N)
        jmin = jnp.min(cand, axis=1, keepdims=True)
        sel = iota_j == jmin
        adj = adj | sel | (iota_i == jmin.reshape(1, N))
        dd = jnp.where(sel, jnp.float32(jnp.inf), dd)

    x = lax.dot_general(z, Win_ref[...], (((1,), (1,)), ((), ())),
                        preferred_element_type=jnp.float32) + bin_ref[...]

    x = _gat_layer(x, adj, W0_ref[...], a0_ref[...], g0_ref[...],
                   be0_ref[...], apply_relu=True)
    x = _gat_layer(x, adj, W1_ref[...], a1_ref[...], g1_ref[...],
                   be1_ref[...], apply_relu=False)
    out_ref[0] = x


@jax.jit
def _run(z, W_in, b_in, W0, a0, g0, be0, W1, a1, g1, be1):
    full = lambda shape: pl.BlockSpec(shape, lambda b: (0,) * len(shape))
    grid_spec = pl.GridSpec(
        grid=(B,),
        in_specs=[
            pl.BlockSpec((1, N, D), lambda b: (b, 0, 0)),
            full((H, D)), full((1, H)),
            full((H, H)), full((1, 2 * H)), full((1, H)), full((1, H)),
            full((H, H)), full((1, 2 * H)), full((1, H)), full((1, H)),
        ],
        out_specs=pl.BlockSpec((1, N, H), lambda b: (b, 0, 0)),
    )
    return pl.pallas_call(
        _body,
        grid_spec=grid_spec,
        out_shape=jax.ShapeDtypeStruct((B, N, H), jnp.float32),
    )(z, W_in, b_in.reshape(1, H), W0, a0, g0.reshape(1, H),
      be0.reshape(1, H), W1, a1, g1.reshape(1, H), be1.reshape(1, H))


def kernel(z, W_in, b_in, W0, a0, g0, be0, W1, a1, g1, be1):
    return _run(z, W_in, b_in, W0, a0, g0, be0, W1, a1, g1, be1)
